# baseline (speedup 1.0000x reference)
import jax
import jax.numpy as jnp
from jax import lax
from jax.experimental import pallas as pl
from jax.experimental.pallas import tpu as pltpu
from jax.experimental.pallas import tpu_sc as plsc

_U = 50000
_I = 50000
_N = _U + _I
_D = 64
_NNZ = 1250000
_B = 4096

_NC = 2
_NS = 16
_L = 16
_NW = _NC * _NS

_BK = 1568
_NBK = 64
_NPAD = _BK * _NBK
_CH = 16 * _BK

_EB = 1024
_E32 = 40960
_NNZ32 = _NW * _E32
_FA = 1024
_SGA = 1552
_DRA = 1536
_RSA = 40 * _FA + _DRA

_AB = 512
_FB = 1536
_SGB = 1920
_DRB = 1920
_RSB = 26 * _FB + _DRB

_BLK = 128
_RING = 3


def _lane(x, e):
    return lax.squeeze(lax.slice(x, (e,), (e + 1,)), (0,))


def _splat(s):
    return jnp.broadcast_to(s, (_L,))


def _part_body(rows, cols, vals, prowA, pcolA, pvalA,
               prowB, pcolB, pvalB, metaB,
               ebr, ebc, ebv, b2r, b2c, b2v,
               stgAr, stgAc, stgAv, stgBr, stgBc, stgBv,
               cntb, flb, mvb, semI, semF):
    cid = lax.axis_index("c")
    sid = lax.axis_index("s")
    wid = sid * _NC + cid
    lanes = lax.broadcasted_iota(jnp.int32, (_L,), 0)
    zi = jnp.zeros((_L,), jnp.int32)
    zf = jnp.zeros((_L,), jnp.float32)
    eb0 = wid * _E32

    cntb[pl.ds(0, _L)] = zi
    flb[pl.ds(0, _L)] = zi

    def fireA(i, h):
        off = eb0 + i * _EB
        pltpu.async_copy(rows.at[pl.ds(off, _EB)], ebr.at[h], semI)
        pltpu.async_copy(cols.at[pl.ds(off, _EB)], ebc.at[h], semI)
        pltpu.async_copy(vals.at[pl.ds(off, _EB)], ebv.at[h], semI)

    def waitA(h):
        pltpu.make_async_copy(rows.at[pl.ds(0, _EB)], ebr.at[h], semI).wait()
        pltpu.make_async_copy(cols.at[pl.ds(0, _EB)], ebc.at[h], semI).wait()
        pltpu.make_async_copy(vals.at[pl.ds(0, _EB)], ebv.at[h], semI).wait()

    fireA(0, 0)

    def chunkA(i2, carry):
        for h in range(2):
            i = i2 * 2 + h
            waitA(h)

            @pl.when(i + 1 < _E32 // _EB)
            def _(i=i, h=h):
                fireA(i + 1, (h + 1) % 2)

            def scanA(g, c2, h=h):
                r = ebr[h, pl.ds(g * _L, _L)]
                c = ebc[h, pl.ds(g * _L, _L)]
                v = ebv[h, pl.ds(g * _L, _L)]
                k = r // _CH
                rel = r - k * _CH
                rank, lastm = plsc.scan_count(k)
                basev = plsc.load_gather(cntb, [k])
                slot = basev + rank - 1
                addr = slot + k * _SGA
                plsc.store_scatter(stgAr, [addr], rel)
                plsc.store_scatter(stgAc, [addr], c)
                plsc.store_scatter(stgAv, [addr], v)
                plsc.addupdate_scatter(cntb, [k], rank, mask=lastm)
                mx = jnp.max(slot + 1)

                @pl.when(mx >= _FA)
                def _():
                    for kb in range(4):
                        cv = cntb[pl.ds(0, _L)]
                        fv = flb[pl.ds(0, _L)]
                        ck = _lane(cv, kb)
                        fk = _lane(fv, kb)

                        @pl.when(ck >= _FA)
                        def _(kb=kb, cv=cv, fv=fv, ck=ck, fk=fk):
                            dst = (wid * 4 + kb) * _RSA + fk * _FA
                            sb = kb * _SGA
                            pltpu.async_copy(stgAr.at[pl.ds(sb, _FA)],
                                             prowA.at[pl.ds(dst, _FA)], semF)
                            pltpu.async_copy(stgAc.at[pl.ds(sb, _FA)],
                                             pcolA.at[pl.ds(dst, _FA)], semF)
                            pltpu.async_copy(stgAv.at[pl.ds(sb, _FA)],
                                             pvalA.at[pl.ds(dst, _FA)], semF)
                            pltpu.make_async_copy(
                                stgAr.at[pl.ds(sb, _FA)],
                                prowA.at[pl.ds(dst, _FA)], semF).wait()
                            pltpu.make_async_copy(
                                stgAc.at[pl.ds(sb, _FA)],
                                pcolA.at[pl.ds(dst, _FA)], semF).wait()
                            pltpu.make_async_copy(
                                stgAv.at[pl.ds(sb, _FA)],
                                pvalA.at[pl.ds(dst, _FA)], semF).wait()
                            tr = stgAr[pl.ds(sb + _FA, _L)]
                            tc = stgAc[pl.ds(sb + _FA, _L)]
                            tv = stgAv[pl.ds(sb + _FA, _L)]
                            stgAr[pl.ds(sb, _L)] = tr
                            stgAc[pl.ds(sb, _L)] = tc
                            stgAv[pl.ds(sb, _L)] = tv
                            cntb[pl.ds(0, _L)] = jnp.where(
                                lanes == kb, cv - _FA, cv)
                            flb[pl.ds(0, _L)] = jnp.where(
                                lanes == kb, fv + 1, fv)
                return c2

            lax.fori_loop(0, _EB // _L, scanA, 0)
        return carry

    lax.fori_loop(0, _E32 // _EB // 2, chunkA, 0)

    nA = []
    cvA = cntb[pl.ds(0, _L)]
    fvA = flb[pl.ds(0, _L)]
    for kb in range(4):
        ck = _lane(cvA, kb)
        fk = _lane(fvA, kb)

        def zpadA(z, c2, kb=kb, ck=ck):
            sb = kb * _SGA
            stgAr[pl.ds(sb + ck + z * _L, _L)] = zi
            stgAc[pl.ds(sb + ck + z * _L, _L)] = zi
            stgAv[pl.ds(sb + ck + z * _L, _L)] = zf
            return c2

        lax.fori_loop(0, _AB // _L, zpadA, 0)
        dst = (wid * 4 + kb) * _RSA + fk * _FA
        sb = kb * _SGA
        pltpu.sync_copy(stgAr.at[pl.ds(sb, _DRA)], prowA.at[pl.ds(dst, _DRA)])
        pltpu.sync_copy(stgAc.at[pl.ds(sb, _DRA)], pcolA.at[pl.ds(dst, _DRA)])
        pltpu.sync_copy(stgAv.at[pl.ds(sb, _DRA)], pvalA.at[pl.ds(dst, _DRA)])
        nA.append((fk * _FA + ck + _AB - 1) // _AB)

    mrows = []
    for kb in range(4):
        cntb[pl.ds(0, _L)] = zi
        flb[pl.ds(0, _L)] = zi
        abase = (wid * 4 + kb) * _RSA

        def fireB(i, h, abase=abase):
            off = abase + i * _AB
            pltpu.async_copy(prowA.at[pl.ds(off, _AB)], b2r.at[h], semI)
            pltpu.async_copy(pcolA.at[pl.ds(off, _AB)], b2c.at[h], semI)
            pltpu.async_copy(pvalA.at[pl.ds(off, _AB)], b2v.at[h], semI)

        def waitB(h):
            pltpu.make_async_copy(prowA.at[pl.ds(0, _AB)], b2r.at[h],
                                  semI).wait()
            pltpu.make_async_copy(pcolA.at[pl.ds(0, _AB)], b2c.at[h],
                                  semI).wait()
            pltpu.make_async_copy(pvalA.at[pl.ds(0, _AB)], b2v.at[h],
                                  semI).wait()

        nAk = nA[kb]

        @pl.when(nAk > 0)
        def _(fireB=fireB):
            fireB(0, 0)

        def blkB(i2, carry, kb=kb, nAk=nAk, fireB=fireB, waitB=waitB):
          for h in range(2):
            i = i2 * 2 + h

            @pl.when(i < nAk)
            def _(i=i, h=h, kb=kb, nAk=nAk, fireB=fireB, waitB=waitB):
              waitB(h)

              @pl.when(i + 1 < nAk)
              def _(i=i, h=h):
                fireB(i + 1, (h + 1) % 2)

              def scanB(g, c2, h=h, kb=kb):
                rel = b2r[h, pl.ds(g * _L, _L)]
                c = b2c[h, pl.ds(g * _L, _L)]
                v = b2v[h, pl.ds(g * _L, _L)]
                k2 = rel // _BK
                rel2 = rel - k2 * _BK
                rank, lastm = plsc.scan_count(k2)
                basev = plsc.load_gather(cntb, [k2])
                slot = basev + rank - 1
                addr = slot + k2 * _SGB
                plsc.store_scatter(stgBr, [addr], rel2)
                plsc.store_scatter(stgBc, [addr], c)
                plsc.store_scatter(stgBv, [addr], v)
                plsc.addupdate_scatter(cntb, [k2], rank, mask=lastm)
                mx = jnp.max(slot + 1)

                @pl.when(mx >= _FB)
                def _(kb=kb):
                    for k16 in range(16):
                        cv = cntb[pl.ds(0, _L)]
                        fv = flb[pl.ds(0, _L)]
                        ck = _lane(cv, k16)
                        fk = _lane(fv, k16)

                        @pl.when(ck >= _FB)
                        def _(k16=k16, cv=cv, fv=fv, ck=ck, fk=fk, kb=kb):
                            g64 = kb * 16 + k16
                            dst = (wid * _NBK + g64) * _RSB + fk * _FB
                            sb = k16 * _SGB
                            pltpu.async_copy(stgBr.at[pl.ds(sb, _FB)],
                                             prowB.at[pl.ds(dst, _FB)], semF)
                            pltpu.async_copy(stgBc.at[pl.ds(sb, _FB)],
                                             pcolB.at[pl.ds(dst, _FB)], semF)
                            pltpu.async_copy(stgBv.at[pl.ds(sb, _FB)],
                                             pvalB.at[pl.ds(dst, _FB)], semF)
                            pltpu.make_async_copy(
                                stgBr.at[pl.ds(sb, _FB)],
                                prowB.at[pl.ds(dst, _FB)], semF).wait()
                            pltpu.make_async_copy(
                                stgBc.at[pl.ds(sb, _FB)],
                                pcolB.at[pl.ds(dst, _FB)], semF).wait()
                            pltpu.make_async_copy(
                                stgBv.at[pl.ds(sb, _FB)],
                                pvalB.at[pl.ds(dst, _FB)], semF).wait()
                            tr = stgBr[pl.ds(sb + _FB, _L)]
                            tc = stgBc[pl.ds(sb + _FB, _L)]
                            tv = stgBv[pl.ds(sb + _FB, _L)]
                            stgBr[pl.ds(sb, _L)] = tr
                            stgBc[pl.ds(sb, _L)] = tc
                            stgBv[pl.ds(sb, _L)] = tv
                            cntb[pl.ds(0, _L)] = jnp.where(
                                lanes == k16, cv - _FB, cv)
                            flb[pl.ds(0, _L)] = jnp.where(
                                lanes == k16, fv + 1, fv)
                return c2

              lax.fori_loop(0, _AB // _L, scanB, 0)
          return carry

        lax.fori_loop(0, (nAk + 1) // 2, blkB, 0)

        cvB = cntb[pl.ds(0, _L)]
        fvB = flb[pl.ds(0, _L)]
        mrow = jnp.zeros((_L,), jnp.int32)
        for k16 in range(16):
            ck = _lane(cvB, k16)
            fk = _lane(fvB, k16)

            def zpadB(z, c2, k16=k16, ck=ck):
                sb = k16 * _SGB
                stgBr[pl.ds(sb + ck + z * _L, _L)] = zi
                stgBc[pl.ds(sb + ck + z * _L, _L)] = zi
                stgBv[pl.ds(sb + ck + z * _L, _L)] = zf
                return c2

            lax.fori_loop(0, 384 // _L, zpadB, 0)
            g64 = kb * 16 + k16
            dst = (wid * _NBK + g64) * _RSB + fk * _FB
            sb = k16 * _SGB
            pltpu.async_copy(stgBr.at[pl.ds(sb, _DRB)],
                             prowB.at[pl.ds(dst, _DRB)], semF)
            pltpu.async_copy(stgBc.at[pl.ds(sb, _DRB)],
                             pcolB.at[pl.ds(dst, _DRB)], semF)
            pltpu.async_copy(stgBv.at[pl.ds(sb, _DRB)],
                             pvalB.at[pl.ds(dst, _DRB)], semF)
            nb3 = (fk * _FB + ck + 383) // 384
            mrow = jnp.where(lanes == k16, _splat(nb3), mrow)
        for k16 in range(16):
            sb = k16 * _SGB
            pltpu.make_async_copy(stgBr.at[pl.ds(sb, _DRB)],
                                  prowB.at[pl.ds(0, _DRB)], semF).wait()
            pltpu.make_async_copy(stgBc.at[pl.ds(sb, _DRB)],
                                  prowB.at[pl.ds(0, _DRB)], semF).wait()
            pltpu.make_async_copy(stgBv.at[pl.ds(sb, _DRB)],
                                  prowB.at[pl.ds(0, _DRB)], semF).wait()
        mrows.append(mrow)

    for kb in range(4):
        mvb[pl.ds(kb * _L, _L)] = mrows[kb]
    pltpu.sync_copy(mvb, metaB.at[wid])


def _spmm_body(eprev, prowB, pcolB, pvalB, metaB, enext,
               mbuf, cbufs, vbufs, rbufs, rowbufs, acc, semE, semG):
    cid = lax.axis_index("c")
    sid = lax.axis_index("s")
    w = sid * _NC + cid
    lanes = lax.broadcasted_iota(jnp.int32, (_L,), 0)
    iota = lanes
    zf = jnp.zeros((_L,), jnp.float32)

    pltpu.sync_copy(metaB, mbuf)

    for sweep in range(2):
        b = sweep * _NW + w
        q16 = b // _L
        lb = b - q16 * _L

        def zacc(z, c2):
            for u in range(8):
                acc[pl.ds((z * 8 + u) * _L, _L)] = zf
            return c2

        lax.fori_loop(0, _BK * _D // _L // 8, zacc, 0)

        def fireE(blk, h, segbase):
            eoff = segbase + blk * _BLK
            pltpu.async_copy(prowB.at[pl.ds(eoff, _BLK)], rbufs[h], semE[h])
            pltpu.async_copy(pcolB.at[pl.ds(eoff, _BLK)], cbufs[h], semE[h])
            pltpu.async_copy(pvalB.at[pl.ds(eoff, _BLK)], vbufs[h], semE[h])

        def waitE(h):
            pltpu.make_async_copy(prowB.at[pl.ds(0, _BLK)], rbufs[h],
                                  semE[h]).wait()
            pltpu.make_async_copy(pcolB.at[pl.ds(0, _BLK)], cbufs[h],
                                  semE[h]).wait()
            pltpu.make_async_copy(pvalB.at[pl.ds(0, _BLK)], vbufs[h],
                                  semE[h]).wait()

        def consume(pb, hp, nb, segbase):
            pltpu.make_async_copy(eprev.at[pl.ds(0, _BLK)], rowbufs[hp],
                                  semG[hp]).wait()

            def scale(g, c2, hp=hp):
                rr = rbufs[hp][pl.ds(g * _L, _L)]
                vv = vbufs[hp][pl.ds(g * _L, _L)]
                for e in range(0, _L, 2):
                    reb0 = _splat(_lane(rr, e))
                    reb1 = _splat(_lane(rr, e + 1))
                    veb0 = _splat(_lane(vv, e))
                    veb1 = _splat(_lane(vv, e + 1))
                    idx0 = reb0 * _D + iota
                    idx1 = reb1 * _D + iota
                    for d in range(_D // _L):
                        x0 = rowbufs[hp][g * _L + e, pl.ds(d * _L, _L)]
                        x1 = rowbufs[hp][g * _L + e + 1, pl.ds(d * _L, _L)]
                        y0 = x0 * veb0
                        y1 = x1 * veb1
                        plsc.addupdate_scatter(acc, [idx0 + d * _L], y0)
                        plsc.addupdate_scatter(acc, [idx1 + d * _L], y1)
                return c2

            lax.fori_loop(0, _BLK // _L, scale, 0)

            @pl.when(pb + _RING < nb)
            def _():
                fireE(pb + _RING, hp, segbase)

        def src_loop(src, carry, sweep=sweep, b=b, q16=q16, lb=lb):
            mv = mbuf[src, pl.ds(q16 * _L, _L)]
            nblk3 = jnp.sum(jnp.where(lanes == lb, mv, 0))
            nb = nblk3 * 3
            segbase = (src * _NBK + b) * _RSB

            @pl.when(nblk3 > 0)
            def _():
                for h in range(_RING):
                    fireE(h, h, segbase)

            def blk_loop(bb, c2):
                for h in range(_RING):
                    blk = bb * _RING + h
                    waitE(h)
                    pltpu.async_copy(eprev.at[cbufs[h]], rowbufs[h], semG[h])
                    hp = (h + _RING - 1) % _RING
                    if h == 0:
                        @pl.when(bb > 0)
                        def _(bb=bb):
                            consume(bb * _RING - 1, _RING - 1, nb, segbase)
                    else:
                        consume(blk - 1, hp, nb, segbase)
                return c2

            lax.fori_loop(0, nblk3, blk_loop, 0)

            @pl.when(nblk3 > 0)
            def _():
                consume(nb - 1, _RING - 1, nb, segbase)
            return carry

        lax.fori_loop(0, _NW, src_loop, 0)

        def out_loop(t, c2, b=b):
            def cpy(f, c3):
                x = acc[pl.ds(t * 98 * _D + f * _L, _L)]
                row = f // 4
                dd = f - row * 4
                rowbufs[0][row, pl.ds(dd * _L, _L)] = x
                return c3

            lax.fori_loop(0, 98 * _D // _L, cpy, 0)
            pltpu.sync_copy(rowbufs[0].at[pl.ds(0, 98)],
                            enext.at[pl.ds(b * _BK + t * 98, 98)])
            return c2

        lax.fori_loop(0, _BK // 98, out_loop, 0)


def _final_body(e0, e1, e2, e3, uid, iid, out,
                ubuf, ibuf, u0, u1, u2, u3, i0, i1, i2, i3, obuf):
    cid = lax.axis_index("c")
    sid = lax.axis_index("s")
    wid = sid * _NC + cid
    nb = _B // _NW
    pltpu.sync_copy(uid.at[pl.ds(wid * nb, nb)], ubuf)
    pltpu.sync_copy(iid.at[pl.ds(wid * nb, nb)], ibuf)

    def adj_body(g, carry):
        ibuf[pl.ds(g * _L, _L)] = ibuf[pl.ds(g * _L, _L)] + _U
        return carry

    lax.fori_loop(0, nb // _L, adj_body, 0)

    for tab, dst in ((e0, u0), (e1, u1), (e2, u2), (e3, u3)):
        pltpu.sync_copy(tab.at[ubuf], dst)
    for tab, dst in ((e0, i0), (e1, i1), (e2, i2), (e3, i3)):
        pltpu.sync_copy(tab.at[ibuf], dst)

    lanes = jax.lax.broadcasted_iota(jnp.int32, (_L,), 0)

    def dot_body(g, carry):
        res = jnp.zeros((_L,), jnp.float32)
        for bb in range(_L):
            b = g * _L + bb
            accv = jnp.zeros((_L,), jnp.float32)
            for d in range(_D // _L):
                sl = pl.ds(d * _L, _L)
                fu = u0[b, sl] + u1[b, sl] + u2[b, sl] + u3[b, sl]
                fi = i0[b, sl] + i1[b, sl] + i2[b, sl] + i3[b, sl]
                accv = accv + fu * fi
            s = jnp.sum(accv) * (1.0 / 16.0)
            res = jnp.where(lanes == bb, jnp.broadcast_to(s, (_L,)), res)
        obuf[pl.ds(g * _L, _L)] = res
        return carry

    lax.fori_loop(0, nb // _L, dot_body, 0)
    pltpu.sync_copy(obuf, out.at[pl.ds(wid * nb, nb)])


_MESH = plsc.VectorSubcoreMesh(core_axis_name="c", subcore_axis_name="s")
_PARAMS = pltpu.CompilerParams(
    use_tc_tiling_on_sc=False, needs_layout_passes=False)
_PTA = _NW * 4 * _RSA
_PTB = _NW * _NBK * _RSB


def _make_part():
    return pl.kernel(
        _part_body,
        out_type=(
            jax.ShapeDtypeStruct((_PTA,), jnp.int32),
            jax.ShapeDtypeStruct((_PTA,), jnp.int32),
            jax.ShapeDtypeStruct((_PTA,), jnp.float32),
            jax.ShapeDtypeStruct((_PTB,), jnp.int32),
            jax.ShapeDtypeStruct((_PTB,), jnp.int32),
            jax.ShapeDtypeStruct((_PTB,), jnp.float32),
            jax.ShapeDtypeStruct((_NW, _NBK), jnp.int32),
        ),
        mesh=_MESH,
        compiler_params=_PARAMS,
        scratch_types=[
            pltpu.VMEM((2, _EB), jnp.int32),
            pltpu.VMEM((2, _EB), jnp.int32),
            pltpu.VMEM((2, _EB), jnp.float32),
            pltpu.VMEM((2, _AB), jnp.int32),
            pltpu.VMEM((2, _AB), jnp.int32),
            pltpu.VMEM((2, _AB), jnp.float32),
            pltpu.VMEM((4 * _SGA,), jnp.int32),
            pltpu.VMEM((4 * _SGA,), jnp.int32),
            pltpu.VMEM((4 * _SGA,), jnp.float32),
            pltpu.VMEM((16 * _SGB,), jnp.int32),
            pltpu.VMEM((16 * _SGB,), jnp.int32),
            pltpu.VMEM((16 * _SGB,), jnp.float32),
            pltpu.VMEM((_L,), jnp.int32),
            pltpu.VMEM((_L,), jnp.int32),
            pltpu.VMEM((_NBK,), jnp.int32),
            pltpu.SemaphoreType.DMA,
            pltpu.SemaphoreType.DMA,
        ],
    )


def _make_spmm():
    return pl.kernel(
        _spmm_body,
        out_type=jax.ShapeDtypeStruct((_NPAD, _D), jnp.float32),
        mesh=_MESH,
        compiler_params=_PARAMS,
        scratch_types=[
            pltpu.VMEM((_NW, _NBK), jnp.int32),
            [pltpu.VMEM((_BLK,), jnp.int32) for _ in range(_RING)],
            [pltpu.VMEM((_BLK,), jnp.float32) for _ in range(_RING)],
            [pltpu.VMEM((_BLK,), jnp.int32) for _ in range(_RING)],
            [pltpu.VMEM((_BLK, _D), jnp.float32) for _ in range(_RING)],
            pltpu.VMEM((_BK * _D,), jnp.float32),
            [pltpu.SemaphoreType.DMA for _ in range(_RING)],
            [pltpu.SemaphoreType.DMA for _ in range(_RING)],
        ],
    )


def _make_final():
    nb = _B // _NW
    return pl.kernel(
        _final_body,
        out_type=jax.ShapeDtypeStruct((_B,), jnp.float32),
        mesh=_MESH,
        compiler_params=_PARAMS,
        scratch_types=[
            pltpu.VMEM((nb,), jnp.int32),
            pltpu.VMEM((nb,), jnp.int32),
            pltpu.VMEM((nb, _D), jnp.float32),
            pltpu.VMEM((nb, _D), jnp.float32),
            pltpu.VMEM((nb, _D), jnp.float32),
            pltpu.VMEM((nb, _D), jnp.float32),
            pltpu.VMEM((nb, _D), jnp.float32),
            pltpu.VMEM((nb, _D), jnp.float32),
            pltpu.VMEM((nb, _D), jnp.float32),
            pltpu.VMEM((nb, _D), jnp.float32),
            pltpu.VMEM((nb,), jnp.float32),
        ],
    )


def kernel(user_ids, item_ids, user_emb, item_emb, adj_row, adj_col, adj_vals):
    e0 = jnp.concatenate([user_emb, item_emb], axis=0)
    e0 = jnp.pad(e0, ((0, _NPAD - _N), (0, 0)))
    pad = _NNZ32 - _NNZ
    rows = jnp.concatenate([adj_row, jnp.arange(pad, dtype=jnp.int32) % _N])
    cols = jnp.pad(adj_col, (0, pad))
    vals = jnp.pad(adj_vals, (0, pad))

    _, _, _, prowB, pcolB, pvalB, metaB = _make_part()(rows, cols, vals)

    spmm = _make_spmm()
    e1 = spmm(e0, prowB, pcolB, pvalB, metaB)
    e2 = spmm(e1, prowB, pcolB, pvalB, metaB)
    e3 = spmm(e2, prowB, pcolB, pvalB, metaB)

    return _make_final()(e0, e1, e2, e3, user_ids, item_ids)


# speedup vs baseline: 5.7337x; 5.7337x over previous
import jax
import jax.numpy as jnp
from jax import lax
from jax.experimental import pallas as pl
from jax.experimental.pallas import tpu as pltpu
from jax.experimental.pallas import tpu_sc as plsc

_U = 50000
_I = 50000
_N = _U + _I
_D = 64
_NNZ = 1250000
_B = 4096

_NC = 2
_NS = 16
_L = 16
_NW = _NC * _NS

_CH = 51200
_NPAD = 2 * _CH
_ZROWS = _CH // _NS

_EB = 4096
_E32 = 40960
_NNZ32 = _NW * _E32
_F = 2048
_STG = 3072
_RS = 20 * _F + _STG

_BLK = 128
_SUB = 128


def _scalar(x):
    return lax.squeeze(lax.slice(x, (0,), (1,)), (0,))


def _part_body(rows, cols, vals, prow, pcol, pval, meta,
               rbufe, cbufe, vbufe, stgr, stgc, stgv, mbuf):
    cid = lax.axis_index("c")
    sid = lax.axis_index("s")
    wid = sid * _NC + cid
    lanes = lax.broadcasted_iota(jnp.int32, (_L,), 0)

    def outer(ob, carry):
        off = wid * _E32 + ob * _EB
        pltpu.sync_copy(rows.at[pl.ds(off, _EB)], rbufe)
        pltpu.sync_copy(cols.at[pl.ds(off, _EB)], cbufe)
        pltpu.sync_copy(vals.at[pl.ds(off, _EB)], vbufe)

        def inner(i, cr):
            r = rbufe[pl.ds(i * _L, _L)]
            c = cbufe[pl.ds(i * _L, _L)]
            v = vbufe[pl.ds(i * _L, _L)]
            k = r // _CH
            rel = r - k * _CH
            cr = list(cr)
            for kk in range(2):
                m = k == kk
                cnt = cr[2 * kk]
                fl = cr[2 * kk + 1]
                plsc.store_compressed(stgr.at[kk, pl.ds(cnt, _L)], rel, mask=m)
                plsc.store_compressed(stgc.at[kk, pl.ds(cnt, _L)], c, mask=m)
                plsc.store_compressed(stgv.at[kk, pl.ds(cnt, _L)], v, mask=m)
                cnt2 = cnt + _scalar(plsc.all_reduce_population_count(m))

                def flush(cc, ff):
                    dst = (wid * 2 + kk) * _RS + ff * _F
                    pltpu.sync_copy(stgr.at[kk, pl.ds(0, _F)],
                                    prow.at[pl.ds(dst, _F)])
                    pltpu.sync_copy(stgc.at[kk, pl.ds(0, _F)],
                                    pcol.at[pl.ds(dst, _F)])
                    pltpu.sync_copy(stgv.at[kk, pl.ds(0, _F)],
                                    pval.at[pl.ds(dst, _F)])
                    tr = stgr[kk, pl.ds(_F, _L)]
                    tc = stgc[kk, pl.ds(_F, _L)]
                    tv = stgv[kk, pl.ds(_F, _L)]
                    stgr[kk, pl.ds(0, _L)] = tr
                    stgc[kk, pl.ds(0, _L)] = tc
                    stgv[kk, pl.ds(0, _L)] = tv
                    return cc - _F, ff + 1

                cnt3, fl3 = lax.cond(cnt2 >= _F, flush,
                                     lambda cc, ff: (cc, ff), cnt2, fl)
                cr[2 * kk] = cnt3
                cr[2 * kk + 1] = fl3
            return tuple(cr)

        return lax.fori_loop(0, _EB // _L, inner, carry)

    z = jnp.int32(0)
    carry = lax.fori_loop(0, _E32 // _EB, outer, (z, z, z, z))

    mvec = jnp.zeros((_L,), jnp.int32)
    zeros_i = jnp.zeros((_L,), jnp.int32)
    zeros_f = jnp.zeros((_L,), jnp.float32)
    for kk in range(2):
        cnt = carry[2 * kk]
        fl = carry[2 * kk + 1]

        def zpad(zi, c3, kk=kk):
            stgr[kk, pl.ds(cnt + zi * _L, _L)] = zeros_i
            stgc[kk, pl.ds(cnt + zi * _L, _L)] = zeros_i
            stgv[kk, pl.ds(cnt + zi * _L, _L)] = zeros_f
            return c3

        lax.fori_loop(0, 256 // _L, zpad, 0)
        dst = (wid * 2 + kk) * _RS + fl * _F
        pltpu.sync_copy(stgr.at[kk, pl.ds(0, _STG)], prow.at[pl.ds(dst, _STG)])
        pltpu.sync_copy(stgc.at[kk, pl.ds(0, _STG)], pcol.at[pl.ds(dst, _STG)])
        pltpu.sync_copy(stgv.at[kk, pl.ds(0, _STG)], pval.at[pl.ds(dst, _STG)])
        nblk2 = (fl * _F + cnt + 255) // 256
        mvec = jnp.where(lanes == kk, jnp.broadcast_to(nblk2, (_L,)), mvec)
    mbuf[pl.ds(0, _L)] = mvec
    pltpu.sync_copy(mbuf, meta.at[wid])


def _spmm_body(eprev, prow, pcol, pval, meta, zrows, enext,
               mbuf, cbufs, vbufs, rbufs, sbufs, rowbufs,
               semE, semG, semS):
    cid = lax.axis_index("c")
    sid = lax.axis_index("s")
    lanes = lax.broadcasted_iota(jnp.int32, (_L,), 0)
    acc = rowbufs[2]

    for sweep in range(1):
        c = cid
        base = c * _CH
        pltpu.sync_copy(zrows, acc.at[pl.ds(sid * _ZROWS, _ZROWS)])
        plsc.subcore_barrier()

        for li in range(2):
            src = sid * _NC + li
            pltpu.sync_copy(meta.at[src], mbuf)
            mv = mbuf[pl.ds(0, _L)]
            nblk2 = jnp.sum(jnp.where(lanes == c, mv, 0))
            nb = nblk2 * 2
            segbase = (src * 2 + c) * _RS

            def fire_edges(b, h):
                eoff = segbase + b * _BLK
                pltpu.async_copy(pcol.at[pl.ds(eoff, _BLK)], cbufs[h], semE[h])
                pltpu.async_copy(pval.at[pl.ds(eoff, _BLK)], vbufs[h], semE[h])
                for u in range(_BLK // _SUB):
                    pltpu.async_copy(
                        prow.at[pl.ds(eoff + u * _SUB, _SUB)],
                        rbufs[h].at[u], semE[h])

            def wait_edges(h):
                pltpu.make_async_copy(
                    pcol.at[pl.ds(0, _BLK)], cbufs[h], semE[h]).wait()
                pltpu.make_async_copy(
                    pval.at[pl.ds(0, _BLK)], vbufs[h], semE[h]).wait()
                for u in range(_BLK // _SUB):
                    pltpu.make_async_copy(
                        prow.at[pl.ds(0, _SUB)], rbufs[h].at[u],
                        semE[h]).wait()

            def drain_scatter(h):
                for u in range(_BLK // _SUB):
                    pltpu.make_async_copy(
                        eprev.at[pl.ds(0, _SUB)],
                        rowbufs[h].at[pl.ds(u * _SUB, _SUB)], semS[h]).wait()

            @pl.when(nblk2 > 0)
            def _():
                fire_edges(0, 0)
                fire_edges(1, 1)

            def blk_body(bb, carry):
                for h in range(2):
                    b = bb * 2 + h

                    @pl.when(bb > 0)
                    def _(h=h):
                        drain_scatter(h)

                    wait_edges(h)
                    descs = [
                        pltpu.async_copy(
                            eprev.at[cbufs[h].at[pl.ds(u * _SUB, _SUB)]],
                            rowbufs[h].at[pl.ds(u * _SUB, _SUB)], semG[h])
                        for u in range(_BLK // _SUB)]
                    for dsc in descs:
                        dsc.wait()

                    def idxcp(q, cr2, h=h):
                        for u in range(_BLK // _SUB):
                            sbufs[h][u, pl.ds(q * _L, _L)] = (
                                rbufs[h][u, pl.ds(q * _L, _L)])
                        return cr2

                    lax.fori_loop(0, _SUB // _L, idxcp, 0)

                    def scale(g, cr2, h=h):
                        e0 = g * _L
                        vv = vbufs[h][pl.ds(e0, _L)]
                        for e in range(_L):
                            vs = lax.squeeze(lax.slice(vv, (e,), (e + 1,)),
                                             (0,))
                            vsp = jnp.broadcast_to(vs, (_L,))
                            vsb = plsc.pack(vsp, vsp,
                                            format=plsc.PackFormat.INTERLEAVED)
                            for d in range(_D // (2 * _L)):
                                sl = pl.ds(d * 2 * _L, 2 * _L)
                                rowbufs[h][e0 + e, sl] = (
                                    rowbufs[h][e0 + e, sl] * vsb)
                        return cr2

                    lax.fori_loop(0, _BLK // _L, scale, 0)
                    for u in range(_BLK // _SUB):
                        pltpu.async_copy(
                            rowbufs[h].at[pl.ds(u * _SUB, _SUB)],
                            acc.at[sbufs[h].at[u]], semS[h], add=True)

                    @pl.when(b + 2 < nb)
                    def _(b=b, h=h):
                        fire_edges(b + 2, h)
                return carry

            lax.fori_loop(0, nblk2, blk_body, 0)

            @pl.when(nblk2 > 0)
            def _():
                drain_scatter(0)
                drain_scatter(1)

        plsc.subcore_barrier()
        pltpu.sync_copy(acc.at[pl.ds(sid * _ZROWS, _ZROWS)],
                        enext.at[pl.ds(base + sid * _ZROWS, _ZROWS)])
        plsc.subcore_barrier()


def _final_body(e0, e1, e2, e3, uid, iid, out,
                ubuf, ibuf, u0, u1, u2, u3, i0, i1, i2, i3, obuf):
    cid = lax.axis_index("c")
    sid = lax.axis_index("s")
    wid = sid * _NC + cid
    nb = _B // _NW
    pltpu.sync_copy(uid.at[pl.ds(wid * nb, nb)], ubuf)
    pltpu.sync_copy(iid.at[pl.ds(wid * nb, nb)], ibuf)

    def adj_body(g, carry):
        ibuf[pl.ds(g * _L, _L)] = ibuf[pl.ds(g * _L, _L)] + _U
        return carry

    lax.fori_loop(0, nb // _L, adj_body, 0)

    for tab, dst in ((e0, u0), (e1, u1), (e2, u2), (e3, u3)):
        pltpu.sync_copy(tab.at[ubuf], dst)
    for tab, dst in ((e0, i0), (e1, i1), (e2, i2), (e3, i3)):
        pltpu.sync_copy(tab.at[ibuf], dst)

    lanes = jax.lax.broadcasted_iota(jnp.int32, (_L,), 0)

    def dot_body(g, carry):
        res = jnp.zeros((_L,), jnp.float32)
        for bb in range(_L):
            b = g * _L + bb
            accv = jnp.zeros((_L,), jnp.float32)
            for d in range(_D // (2 * _L)):
                sl = pl.ds(d * 2 * _L, 2 * _L)
                fu0 = jnp.zeros((_L,), jnp.float32)
                fu1 = jnp.zeros((_L,), jnp.float32)
                fi0 = jnp.zeros((_L,), jnp.float32)
                fi1 = jnp.zeros((_L,), jnp.float32)
                for tab in (u0, u1, u2, u3):
                    a, bb2 = plsc.unpack(tab[b, sl],
                                         format=plsc.PackFormat.INTERLEAVED)
                    fu0 = fu0 + a
                    fu1 = fu1 + bb2
                for tab in (i0, i1, i2, i3):
                    a, bb2 = plsc.unpack(tab[b, sl],
                                         format=plsc.PackFormat.INTERLEAVED)
                    fi0 = fi0 + a
                    fi1 = fi1 + bb2
                accv = accv + fu0 * fi0 + fu1 * fi1
            s = jnp.sum(accv) * (1.0 / 16.0)
            res = jnp.where(lanes == bb, jnp.broadcast_to(s, (_L,)), res)
        obuf[pl.ds(g * _L, _L)] = res
        return carry

    lax.fori_loop(0, nb // _L, dot_body, 0)
    pltpu.sync_copy(obuf, out.at[pl.ds(wid * nb, nb)])


_MESH = plsc.VectorSubcoreMesh(core_axis_name="c", subcore_axis_name="s")
_PARAMS = pltpu.CompilerParams(
    use_tc_tiling_on_sc=False, needs_layout_passes=False)
_PTOT = _NW * 2 * _RS


def _make_part():
    return pl.kernel(
        _part_body,
        out_type=(
            jax.ShapeDtypeStruct((_PTOT,), jnp.int32),
            jax.ShapeDtypeStruct((_PTOT,), jnp.int32),
            jax.ShapeDtypeStruct((_PTOT,), jnp.float32),
            jax.ShapeDtypeStruct((_NW, _L), jnp.int32),
        ),
        mesh=_MESH,
        compiler_params=_PARAMS,
        scratch_types=[
            pltpu.VMEM((_EB,), jnp.int32),
            pltpu.VMEM((_EB,), jnp.int32),
            pltpu.VMEM((_EB,), jnp.float32),
            pltpu.VMEM((2, _STG), jnp.int32),
            pltpu.VMEM((2, _STG), jnp.int32),
            pltpu.VMEM((2, _STG), jnp.float32),
            pltpu.VMEM((_L,), jnp.int32),
        ],
    )


def _make_spmm():
    nsub = _BLK // _SUB
    return pl.kernel(
        _spmm_body,
        out_type=jax.ShapeDtypeStruct((_NPAD, _D), jnp.bfloat16),
        mesh=_MESH,
        compiler_params=_PARAMS,
        scratch_types=[
            pltpu.VMEM((_L,), jnp.int32),
            [pltpu.VMEM((_BLK,), jnp.int32) for _ in range(2)],
            [pltpu.VMEM((_BLK,), jnp.float32) for _ in range(2)],
            [pltpu.VMEM((nsub, _SUB), jnp.int32) for _ in range(2)],
            [pltpu.VMEM((nsub, _SUB), jnp.int32) for _ in range(2)],
            [pltpu.VMEM((_BLK, _D), jnp.bfloat16) for _ in range(2)]
            + [pltpu.VMEM_SHARED((_CH, _D), jnp.bfloat16)],
            [pltpu.SemaphoreType.DMA for _ in range(2)],
            [pltpu.SemaphoreType.DMA for _ in range(2)],
            [pltpu.SemaphoreType.DMA for _ in range(2)],
        ],
    )


def _make_final():
    nb = _B // _NW
    return pl.kernel(
        _final_body,
        out_type=jax.ShapeDtypeStruct((_B,), jnp.float32),
        mesh=_MESH,
        compiler_params=_PARAMS,
        scratch_types=[
            pltpu.VMEM((nb,), jnp.int32),
            pltpu.VMEM((nb,), jnp.int32),
            pltpu.VMEM((nb, _D), jnp.bfloat16),
            pltpu.VMEM((nb, _D), jnp.bfloat16),
            pltpu.VMEM((nb, _D), jnp.bfloat16),
            pltpu.VMEM((nb, _D), jnp.bfloat16),
            pltpu.VMEM((nb, _D), jnp.bfloat16),
            pltpu.VMEM((nb, _D), jnp.bfloat16),
            pltpu.VMEM((nb, _D), jnp.bfloat16),
            pltpu.VMEM((nb, _D), jnp.bfloat16),
            pltpu.VMEM((nb,), jnp.float32),
        ],
    )


def kernel(user_ids, item_ids, user_emb, item_emb, adj_row, adj_col, adj_vals):
    e0 = jnp.concatenate([user_emb, item_emb], axis=0).astype(jnp.bfloat16)
    e0 = jnp.pad(e0, ((0, _NPAD - _N), (0, 0)))
    pad = _NNZ32 - _NNZ
    rows = jnp.concatenate([adj_row, jnp.arange(pad, dtype=jnp.int32) % _N])
    cols = jnp.pad(adj_col, (0, pad))
    vals = jnp.pad(adj_vals, (0, pad))
    zrows = jnp.zeros((_ZROWS, _D), jnp.bfloat16)

    prow, pcol, pval, meta = _make_part()(rows, cols, vals)

    spmm = _make_spmm()
    e1 = spmm(e0, prow, pcol, pval, meta, zrows)
    e2 = spmm(e1, prow, pcol, pval, meta, zrows)
    e3 = spmm(e2, prow, pcol, pval, meta, zrows)

    return _make_final()(e0, e1, e2, e3, user_ids, item_ids)
